# Initial kernel scaffold; baseline (speedup 1.0000x reference)
import jax
import jax.numpy as jnp
from jax.experimental import pallas as pl


def kernel(peptide_x, mhc_x, peptide_emb_w, mhc_emb_w):
    raise NotImplementedError("write your pallas kernel here")


# speedup vs baseline: 1.2721x; 1.2721x over previous
import functools

import jax
import jax.numpy as jnp
from jax import lax
from jax.experimental import pallas as pl
from jax.experimental.pallas import tpu as pltpu
from jax.experimental.pallas import tpu_sc as plsc

VOCAB = 21
EMB = 128
BATCH = 16384
PEP_LEN = 21
MHC_LEN = 34
PAD = 3

NC, NS = 2, 16
NW = NC * NS
CHUNK = 128

PEP_ROWS = BATCH * PEP_LEN
MHC_ROWS = BATCH * MHC_LEN
PEP_CH_W = PEP_ROWS // (NW * CHUNK)
MHC_CH_W = MHC_ROWS // (NW * CHUNK)
assert PEP_CH_W * NW * CHUNK == PEP_ROWS
assert MHC_CH_W * NW * CHUNK == MHC_ROWS

_mesh = plsc.VectorSubcoreMesh(
    core_axis_name="c", subcore_axis_name="s", num_cores=NC, num_subcores=NS)


@functools.partial(
    pl.kernel,
    out_type=(
        jax.ShapeDtypeStruct((PEP_ROWS, EMB), jnp.float32),
        jax.ShapeDtypeStruct((MHC_ROWS, EMB), jnp.float32),
    ),
    mesh=_mesh,
    scratch_types=[
        pltpu.VMEM((PEP_CH_W, CHUNK), jnp.int32),
        pltpu.VMEM((MHC_CH_W, CHUNK), jnp.int32),
        pltpu.VMEM((CHUNK, EMB), jnp.float32),
        pltpu.SemaphoreType.DMA,
    ],
)
def _sc_gather(pep_idx_hbm, mhc_idx_hbm, pep_w_hbm, mhc_w_hbm,
               pep_out_hbm, mhc_out_hbm,
               pep_idx_v, mhc_idx_v, rows_v, gsem):
    wid = lax.axis_index("s") * NC + lax.axis_index("c")

    pltpu.sync_copy(pep_idx_hbm.at[wid], pep_idx_v)
    pltpu.sync_copy(mhc_idx_hbm.at[wid], mhc_idx_v)

    def run(table_hbm, idx_v, out_hbm, n_ch, base_ch):
        def body(g, carry):
            pltpu.async_copy(table_hbm.at[idx_v.at[g]], rows_v, gsem).wait()
            pltpu.sync_copy(
                rows_v, out_hbm.at[pl.ds((base_ch + g) * CHUNK, CHUNK)])
            return carry
        lax.fori_loop(0, n_ch, body, 0, unroll=False)

    run(pep_w_hbm, pep_idx_v, pep_out_hbm, PEP_CH_W, wid * PEP_CH_W)
    run(mhc_w_hbm, mhc_idx_v, mhc_out_hbm, MHC_CH_W, wid * MHC_CH_W)


def _mask_body(x_ref, o_ref):
    o_ref[...] = x_ref[...] != 0


_mask_call = pl.pallas_call(
    _mask_body,
    out_shape=jax.ShapeDtypeStruct((BATCH, PEP_LEN - 2 * PAD), jnp.bool_),
)


def kernel(peptide_x, mhc_x, peptide_emb_w, mhc_emb_w):
    pep_idx = peptide_x.reshape(NW, PEP_CH_W, CHUNK).astype(jnp.int32)
    mhc_idx = mhc_x.reshape(NW, MHC_CH_W, CHUNK).astype(jnp.int32)
    pep_flat, mhc_flat = _sc_gather(pep_idx, mhc_idx, peptide_emb_w, mhc_emb_w)
    pep_emb = pep_flat.reshape(BATCH, PEP_LEN, EMB)
    mhc_emb = mhc_flat.reshape(BATCH, MHC_LEN, EMB)
    masks = _mask_call(peptide_x[:, PAD:PEP_LEN - PAD])
    return (pep_emb, mhc_emb, masks)


# speedup vs baseline: 1.2791x; 1.0055x over previous
import functools

import jax
import jax.numpy as jnp
from jax import lax
from jax.experimental import pallas as pl
from jax.experimental.pallas import tpu as pltpu
from jax.experimental.pallas import tpu_sc as plsc

VOCAB = 21
EMB = 128
BATCH = 16384
PEP_LEN = 21
MHC_LEN = 34
PAD = 3

NC, NS = 2, 16
NW = NC * NS
CHUNK = 128
NBUF = 4

PEP_ROWS = BATCH * PEP_LEN
MHC_ROWS = BATCH * MHC_LEN
PEP_CH_W = PEP_ROWS // (NW * CHUNK)
MHC_CH_W = MHC_ROWS // (NW * CHUNK)
assert PEP_CH_W * NW * CHUNK == PEP_ROWS
assert MHC_CH_W * NW * CHUNK == MHC_ROWS

_mesh = plsc.VectorSubcoreMesh(
    core_axis_name="c", subcore_axis_name="s", num_cores=NC, num_subcores=NS)


@functools.partial(
    pl.kernel,
    out_type=(
        jax.ShapeDtypeStruct((PEP_ROWS, EMB), jnp.float32),
        jax.ShapeDtypeStruct((MHC_ROWS, EMB), jnp.float32),
    ),
    mesh=_mesh,
    scratch_types=[
        pltpu.VMEM((PEP_CH_W, CHUNK), jnp.int32),
        pltpu.VMEM((MHC_CH_W, CHUNK), jnp.int32),
        pltpu.VMEM((NBUF, CHUNK, EMB), jnp.float32),
        pltpu.SemaphoreType.DMA((NBUF,)),
        pltpu.SemaphoreType.DMA((NBUF,)),
    ],
)
def _sc_gather(pep_idx_hbm, mhc_idx_hbm, pep_w_hbm, mhc_w_hbm,
               pep_out_hbm, mhc_out_hbm,
               pep_idx_v, mhc_idx_v, rows_v, gsem, ssem):
    wid = lax.axis_index("s") * NC + lax.axis_index("c")

    pltpu.sync_copy(pep_idx_hbm.at[wid], pep_idx_v)
    pltpu.sync_copy(mhc_idx_hbm.at[wid], mhc_idx_v)

    def run(table_hbm, idx_v, out_hbm, n_ch, base_ch):
        def gstart(c, b):
            pltpu.async_copy(table_hbm.at[idx_v.at[c]], rows_v.at[b],
                             gsem.at[b])

        def gwait(c, b):
            pltpu.make_async_copy(table_hbm.at[idx_v.at[c]], rows_v.at[b],
                                  gsem.at[b]).wait()

        def sstart(c, b):
            pltpu.async_copy(
                rows_v.at[b], out_hbm.at[pl.ds((base_ch + c) * CHUNK, CHUNK)],
                ssem.at[b])

        def swait(b):
            pltpu.make_async_copy(
                rows_v.at[b], out_hbm.at[pl.ds(0, CHUNK)], ssem.at[b]).wait()

        gstart(0, 0)
        for b in range(NBUF):
            bn = (b + 1) % NBUF
            if b == NBUF - 1:
                swait(bn)
            gstart(b + 1, bn)
            gwait(b, b)
            sstart(b, b)

        @pl.loop(NBUF, n_ch - NBUF, step=NBUF)
        def _mid(g0):
            for b in range(NBUF):
                c = g0 + b
                bn = (b + 1) % NBUF
                swait(bn)
                gstart(c + 1, bn)
                gwait(c, b)
                sstart(c, b)

        for b in range(NBUF):
            c = n_ch - NBUF + b
            bn = (b + 1) % NBUF
            if b < NBUF - 1:
                swait(bn)
                gstart(c + 1, bn)
            gwait(c, b)
            sstart(c, b)
        for b in range(NBUF):
            swait(b)

    run(pep_w_hbm, pep_idx_v, pep_out_hbm, PEP_CH_W, wid * PEP_CH_W)
    run(mhc_w_hbm, mhc_idx_v, mhc_out_hbm, MHC_CH_W, wid * MHC_CH_W)


def _mask_body(x_ref, o_ref):
    o_ref[...] = x_ref[...] != 0


_mask_call = pl.pallas_call(
    _mask_body,
    out_shape=jax.ShapeDtypeStruct((BATCH, PEP_LEN - 2 * PAD), jnp.bool_),
)


def kernel(peptide_x, mhc_x, peptide_emb_w, mhc_emb_w):
    pep_idx = peptide_x.reshape(NW, PEP_CH_W, CHUNK).astype(jnp.int32)
    mhc_idx = mhc_x.reshape(NW, MHC_CH_W, CHUNK).astype(jnp.int32)
    pep_flat, mhc_flat = _sc_gather(pep_idx, mhc_idx, peptide_emb_w, mhc_emb_w)
    pep_emb = pep_flat.reshape(BATCH, PEP_LEN, EMB)
    mhc_emb = mhc_flat.reshape(BATCH, MHC_LEN, EMB)
    masks = _mask_call(peptide_x[:, PAD:PEP_LEN - PAD])
    return (pep_emb, mhc_emb, masks)


# speedup vs baseline: 2.4852x; 1.9430x over previous
import functools

import jax
import jax.numpy as jnp
from jax import lax
from jax.experimental import pallas as pl
from jax.experimental.pallas import tpu as pltpu
from jax.experimental.pallas import tpu_sc as plsc

VOCAB = 21
VOCAB_PAD = 24
EMB = 128
BATCH = 16384
PEP_LEN = 21
MHC_LEN = 34
PAD = 3

NC, NS = 2, 16
NW = NC * NS
ROWS_W = BATCH // NW
L16 = 16
NSEG = EMB // L16

PEP_BLK = 8
MHC_BLK = 4
SUPER = 64
BUF_ROWS = PEP_BLK * PEP_LEN

PEP_ROWS = BATCH * PEP_LEN
MHC_ROWS = BATCH * MHC_LEN

_mesh = plsc.VectorSubcoreMesh(
    core_axis_name="c", subcore_axis_name="s", num_cores=NC, num_subcores=NS)


@functools.partial(
    pl.kernel,
    out_type=(
        jax.ShapeDtypeStruct((PEP_ROWS, EMB), jnp.float32),
        jax.ShapeDtypeStruct((MHC_ROWS, EMB), jnp.float32),
    ),
    mesh=_mesh,
    scratch_types=[
        pltpu.VMEM((SUPER, PEP_LEN), jnp.int32),
        pltpu.VMEM((SUPER, MHC_LEN), jnp.int32),
        pltpu.VMEM((VOCAB_PAD, EMB), jnp.float32),
        pltpu.VMEM((VOCAB_PAD, EMB), jnp.float32),
        pltpu.VMEM((2, BUF_ROWS, EMB), jnp.float32),
        pltpu.SemaphoreType.DMA((2,)),
    ],
)
def _sc_gather(pep_idx_hbm, mhc_idx_hbm, pep_w_hbm, mhc_w_hbm,
               pep_out_hbm, mhc_out_hbm,
               pep_idx_v, mhc_idx_v, pep_tab_v, mhc_tab_v, rows_v, ssem):
    wid = lax.axis_index("s") * NC + lax.axis_index("c")

    pltpu.sync_copy(pep_w_hbm, pep_tab_v)
    pltpu.sync_copy(mhc_w_hbm, mhc_tab_v)

    def run(tab_v, idx_hbm, idx_v, out_hbm, seq_len, blk_rows, out_base):
        rows_per_blk = blk_rows * seq_len
        bps = SUPER // blk_rows
        n_super = ROWS_W // SUPER

        seg_offs = []
        off = 0
        while off + L16 < seq_len:
            seg_offs.append(off)
            off += L16
        seg_offs.append(seq_len - L16)

        def load_super(s):
            pltpu.sync_copy(
                idx_hbm.at[pl.ds(wid * ROWS_W + s * SUPER, SUPER)], idx_v)

        def fill(lblk, b):
            @pl.loop(0, blk_rows)
            def _row(j):
                r = lblk * blk_rows + j
                segs = [idx_v[r, pl.ds(o, L16)] for o in seg_offs]
                for k in range(seq_len):
                    si = min(k // L16, len(seg_offs) - 1)
                    t = segs[si][k - seg_offs[si]]
                    dst = j * seq_len + k
                    for c in range(NSEG):
                        rows_v[b, dst, pl.ds(c * L16, L16)] = (
                            tab_v[t, pl.ds(c * L16, L16)])

        def sstart(gblk, b):
            pltpu.async_copy(
                rows_v.at[b, pl.ds(0, rows_per_blk)],
                out_hbm.at[pl.ds(out_base + gblk * rows_per_blk,
                                 rows_per_blk)],
                ssem.at[b])

        def swait(b):
            pltpu.make_async_copy(
                rows_v.at[b, pl.ds(0, rows_per_blk)],
                out_hbm.at[pl.ds(0, rows_per_blk)], ssem.at[b]).wait()

        for b in range(2):
            sstart(b, b)

        @pl.loop(0, n_super)
        def _s(s):
            load_super(s)

            @pl.loop(0, bps, step=2)
            def _i(i):
                for b in range(2):
                    swait(b)
                    fill(i + b, b)
                    sstart(s * bps + i + b, b)

        for b in range(2):
            swait(b)

    run(pep_tab_v, pep_idx_hbm, pep_idx_v, pep_out_hbm, PEP_LEN, PEP_BLK,
        wid * ROWS_W * PEP_LEN)
    run(mhc_tab_v, mhc_idx_hbm, mhc_idx_v, mhc_out_hbm, MHC_LEN, MHC_BLK,
        wid * ROWS_W * MHC_LEN)


def _mask_body(x_ref, o_ref):
    o_ref[...] = x_ref[...] != 0


_mask_call = pl.pallas_call(
    _mask_body,
    out_shape=jax.ShapeDtypeStruct((BATCH, PEP_LEN - 2 * PAD), jnp.bool_),
)


def kernel(peptide_x, mhc_x, peptide_emb_w, mhc_emb_w):
    pad = ((0, VOCAB_PAD - VOCAB), (0, 0))
    pep_flat, mhc_flat = _sc_gather(
        peptide_x.astype(jnp.int32), mhc_x.astype(jnp.int32),
        jnp.pad(peptide_emb_w, pad), jnp.pad(mhc_emb_w, pad))
    pep_emb = pep_flat.reshape(BATCH, PEP_LEN, EMB)
    mhc_emb = mhc_flat.reshape(BATCH, MHC_LEN, EMB)
    masks = _mask_call(peptide_x[:, PAD:PEP_LEN - PAD])
    return (pep_emb, mhc_emb, masks)


# speedup vs baseline: 2.4866x; 1.0006x over previous
import functools

import jax
import jax.numpy as jnp
from jax import lax
from jax.experimental import pallas as pl
from jax.experimental.pallas import tpu as pltpu
from jax.experimental.pallas import tpu_sc as plsc

VOCAB = 21
VOCAB_PAD = 24
EMB = 128
BATCH = 16384
PEP_LEN = 21
MHC_LEN = 34
PAD = 3

NC, NS = 2, 16
NW = NC * NS
ROWS_W = BATCH // NW
L16 = 16
NSEG = EMB // L16

PEP_BLK = 8
MHC_BLK = 4
SUPER = 64
BUF_ROWS = PEP_BLK * PEP_LEN

PEP_ROWS = BATCH * PEP_LEN
MHC_ROWS = BATCH * MHC_LEN
IDX_W = 128

_mesh = plsc.VectorSubcoreMesh(
    core_axis_name="c", subcore_axis_name="s", num_cores=NC, num_subcores=NS)


@functools.partial(
    pl.kernel,
    out_type=(
        jax.ShapeDtypeStruct((PEP_ROWS, EMB), jnp.float32),
        jax.ShapeDtypeStruct((MHC_ROWS, EMB), jnp.float32),
    ),
    mesh=_mesh,
    scratch_types=[
        pltpu.VMEM((SUPER, IDX_W), jnp.int32),
        pltpu.VMEM((VOCAB_PAD, EMB), jnp.float32),
        pltpu.VMEM((VOCAB_PAD, EMB), jnp.float32),
        pltpu.VMEM((2, BUF_ROWS, EMB), jnp.float32),
        pltpu.SemaphoreType.DMA((2,)),
    ],
)
def _sc_gather(idx_hbm, pep_w_hbm, mhc_w_hbm,
               pep_out_hbm, mhc_out_hbm,
               idx_v, pep_tab_v, mhc_tab_v, rows_v, ssem):
    wid = lax.axis_index("s") * NC + lax.axis_index("c")

    pltpu.sync_copy(pep_w_hbm, pep_tab_v)
    pltpu.sync_copy(mhc_w_hbm, mhc_tab_v)

    def run(tab_v, col_base, out_hbm, seq_len, blk_rows, out_base):
        rows_per_blk = blk_rows * seq_len
        bps = SUPER // blk_rows
        n_super = ROWS_W // SUPER

        seg_offs = [col_base + o for o in range(0, seq_len, L16)]

        def load_super(s):
            pltpu.sync_copy(
                idx_hbm.at[pl.ds(wid * ROWS_W + s * SUPER, SUPER)], idx_v)

        def fill(lblk, b):
            @pl.loop(0, blk_rows)
            def _row(j):
                r = lblk * blk_rows + j
                segs = [idx_v[r, pl.ds(o, L16)] for o in seg_offs]
                for k in range(seq_len):
                    si = k // L16
                    t = segs[si][k % L16]
                    dst = j * seq_len + k
                    for c in range(NSEG):
                        rows_v[b, dst, pl.ds(c * L16, L16)] = (
                            tab_v[t, pl.ds(c * L16, L16)])

        def sstart(gblk, b):
            pltpu.async_copy(
                rows_v.at[b, pl.ds(0, rows_per_blk)],
                out_hbm.at[pl.ds(out_base + gblk * rows_per_blk,
                                 rows_per_blk)],
                ssem.at[b])

        def swait(b):
            pltpu.make_async_copy(
                rows_v.at[b, pl.ds(0, rows_per_blk)],
                out_hbm.at[pl.ds(0, rows_per_blk)], ssem.at[b]).wait()

        for b in range(2):
            sstart(b, b)

        @pl.loop(0, n_super)
        def _s(s):
            load_super(s)

            @pl.loop(0, bps, step=2)
            def _i(i):
                for b in range(2):
                    swait(b)
                    fill(i + b, b)
                    sstart(s * bps + i + b, b)

        for b in range(2):
            swait(b)

    run(pep_tab_v, 0, pep_out_hbm, PEP_LEN, PEP_BLK,
        wid * ROWS_W * PEP_LEN)
    run(mhc_tab_v, PEP_LEN, mhc_out_hbm, MHC_LEN, MHC_BLK,
        wid * ROWS_W * MHC_LEN)


def _mask_body(x_ref, o_ref):
    o_ref[...] = x_ref[...] != 0


_mask_call = pl.pallas_call(
    _mask_body,
    out_shape=jax.ShapeDtypeStruct((BATCH, PEP_LEN - 2 * PAD), jnp.bool_),
)


def kernel(peptide_x, mhc_x, peptide_emb_w, mhc_emb_w):
    pad = ((0, VOCAB_PAD - VOCAB), (0, 0))
    idx_packed = jnp.pad(
        jnp.concatenate(
            [peptide_x.astype(jnp.int32), mhc_x.astype(jnp.int32)], axis=1),
        ((0, 0), (0, IDX_W - PEP_LEN - MHC_LEN)))
    pep_flat, mhc_flat = _sc_gather(
        idx_packed, jnp.pad(peptide_emb_w, pad), jnp.pad(mhc_emb_w, pad))
    pep_emb = pep_flat.reshape(BATCH, PEP_LEN, EMB)
    mhc_emb = mhc_flat.reshape(BATCH, MHC_LEN, EMB)
    masks = _mask_call(peptide_x[:, PAD:PEP_LEN - PAD])
    return (pep_emb, mhc_emb, masks)


# speedup vs baseline: 3.5201x; 1.4156x over previous
import functools

import jax
import jax.numpy as jnp
from jax import lax
from jax.experimental import pallas as pl
from jax.experimental.pallas import tpu as pltpu
from jax.experimental.pallas import tpu_sc as plsc

VOCAB = 21
VOCAB_PAD = 24
EMB = 128
BATCH = 16384
PEP_LEN = 21
MHC_LEN = 34
PAD = 3

NC, NS = 2, 16
NW = NC * NS
ROWS_W = BATCH // NW
L16 = 16
NSEG = EMB // L16

PEP_BLK = 8
MHC_BLK = 4
SUPER = 64
BUF_ROWS = PEP_BLK * PEP_LEN

PEP_ROWS = BATCH * PEP_LEN
MHC_ROWS = BATCH * MHC_LEN
IDX_W = 128

_mesh = plsc.VectorSubcoreMesh(
    core_axis_name="c", subcore_axis_name="s", num_cores=NC, num_subcores=NS)


@functools.partial(
    pl.kernel,
    out_type=(
        jax.ShapeDtypeStruct((BATCH, PEP_LEN, EMB), jnp.float32),
        jax.ShapeDtypeStruct((BATCH, MHC_LEN, EMB), jnp.float32),
    ),
    mesh=_mesh,
    scratch_types=[
        pltpu.VMEM((SUPER, IDX_W), jnp.int32),
        pltpu.VMEM((VOCAB_PAD, EMB), jnp.float32),
        pltpu.VMEM((VOCAB_PAD, EMB), jnp.float32),
        pltpu.VMEM((2, PEP_BLK, PEP_LEN, EMB), jnp.float32),
        pltpu.VMEM((2, MHC_BLK, MHC_LEN, EMB), jnp.float32),
        pltpu.SemaphoreType.DMA((2,)),
    ],
)
def _sc_gather(idx_hbm, pep_w_hbm, mhc_w_hbm,
               pep_out_hbm, mhc_out_hbm,
               idx_v, pep_tab_v, mhc_tab_v, pep_rows_v, mhc_rows_v, ssem):
    wid = lax.axis_index("s") * NC + lax.axis_index("c")

    pltpu.sync_copy(pep_w_hbm, pep_tab_v)
    pltpu.sync_copy(mhc_w_hbm, mhc_tab_v)

    def run(tab_v, rows_v, col_base, out_hbm, seq_len, blk_rows):
        bps = SUPER // blk_rows
        n_super = ROWS_W // SUPER
        out_base = wid * ROWS_W

        seg_offs = [col_base + o for o in range(0, seq_len, L16)]

        def load_super(s):
            pltpu.sync_copy(
                idx_hbm.at[pl.ds(wid * ROWS_W + s * SUPER, SUPER)], idx_v)

        def fill(lblk, b):
            @pl.loop(0, blk_rows)
            def _row(j):
                r = lblk * blk_rows + j
                segs = [idx_v[r, pl.ds(o, L16)] for o in seg_offs]
                for k in range(seq_len):
                    si = k // L16
                    t = segs[si][k % L16]
                    for c in range(NSEG):
                        rows_v[b, j, k, pl.ds(c * L16, L16)] = (
                            tab_v[t, pl.ds(c * L16, L16)])

        def sstart(gblk, b):
            pltpu.async_copy(
                rows_v.at[b],
                out_hbm.at[pl.ds(out_base + gblk * blk_rows, blk_rows)],
                ssem.at[b])

        def swait(b):
            pltpu.make_async_copy(
                rows_v.at[b], out_hbm.at[pl.ds(0, blk_rows)],
                ssem.at[b]).wait()

        for b in range(2):
            sstart(b, b)

        @pl.loop(0, n_super)
        def _s(s):
            load_super(s)

            @pl.loop(0, bps, step=2)
            def _i(i):
                for b in range(2):
                    swait(b)
                    fill(i + b, b)
                    sstart(s * bps + i + b, b)

        for b in range(2):
            swait(b)

    run(pep_tab_v, pep_rows_v, 0, pep_out_hbm, PEP_LEN, PEP_BLK)
    run(mhc_tab_v, mhc_rows_v, PEP_LEN, mhc_out_hbm, MHC_LEN, MHC_BLK)


def _mask_body(x_ref, o_ref):
    o_ref[...] = x_ref[...] != 0


_mask_call = pl.pallas_call(
    _mask_body,
    out_shape=jax.ShapeDtypeStruct((BATCH, PEP_LEN - 2 * PAD), jnp.bool_),
)


def kernel(peptide_x, mhc_x, peptide_emb_w, mhc_emb_w):
    pad = ((0, VOCAB_PAD - VOCAB), (0, 0))
    idx_packed = jnp.pad(
        jnp.concatenate(
            [peptide_x.astype(jnp.int32), mhc_x.astype(jnp.int32)], axis=1),
        ((0, 0), (0, IDX_W - PEP_LEN - MHC_LEN)))
    pep_emb, mhc_emb = _sc_gather(
        idx_packed, jnp.pad(peptide_emb_w, pad), jnp.pad(mhc_emb_w, pad))
    masks = _mask_call(peptide_x[:, PAD:PEP_LEN - PAD])
    return (pep_emb, mhc_emb, masks)
